# baseline (speedup 1.0000x reference)
import functools
import jax
import jax.numpy as jnp
from jax import lax
from jax.experimental import pallas as pl
from jax.experimental.pallas import tpu as pltpu
from jax.experimental.pallas import tpu_sc as plsc

_K = 5
_R = 1024
_L = 16
_NPAD = 20480
_NW = 16
_BPW = _R // _NW
_NEG = -1e30


def _merge16(u, v):
    return lax.sort(jnp.maximum(u, lax.rev(v, (0,))))


def _sc_body(b_hbm, d_hbm, outa_hbm, outb_hbm, sums_hbm,
             bv, dv, keys, bks, tab, tab2, tab3, tab4, scn, blk, sumv, tot):
    c = lax.axis_index("c")
    s = lax.axis_index("s")
    is_a = c == 0
    lo = s * _BPW
    lanes = lax.iota(jnp.int32, _L)
    neg_row = jnp.full((_L,), _NEG, jnp.float32)

    pltpu.sync_copy(b_hbm, bv)
    pltpu.sync_copy(d_hbm, dv)

    def mm_body(i, carry):
        mb, md = carry
        return (jnp.minimum(mb, bv[pl.ds(i * _L, _L)]),
                jnp.maximum(md, dv[pl.ds(i * _L, _L)]))
    mb, md = lax.fori_loop(0, _NPAD // _L, mm_body,
                           (jnp.full((_L,), 1e30, jnp.float32), neg_row))
    minb = lax.sort(mb)[0]
    maxd = lax.sort(md)[_L - 1]
    spanv = jnp.full((_L,), maxd - minb, jnp.float32)
    inv_dtv = jnp.where(spanv > 0, jnp.float32(_R - 1) / spanv,
                        jnp.zeros((_L,), jnp.float32))

    half = _NPAD // (2 * _L)
    base1 = _NPAD // 2 + _L

    def f_chunk(i, off, base):
        bb = bv[pl.ds(i * _L, _L)]
        dd = dv[pl.ds(i * _L, _L)]
        u = ((bb + dd) * 0.5 - minb) * inv_dtv
        ti = u.astype(jnp.int32)
        bk = jnp.where(ti.astype(jnp.float32) < u, ti + 1, ti)
        bk = jnp.clip(bk, 0, _R - 1)
        key = jnp.where(is_a, dd, -bb)
        msk = (bk >= lo) & (bk < lo + _BPW)
        mi = msk.astype(jnp.int32)
        cum = plsc.cumsum(mi)
        pos = base + off + cum - mi
        plsc.store_scatter(keys, [pos], key, mask=msk)
        plsc.store_scatter(bks, [pos], bk, mask=msk)
        return off + cum[_L - 1]

    def f_body(i, offs):
        return (f_chunk(i, offs[0], 0), f_chunk(i + half, offs[1], base1))
    cnt0, cnt1 = lax.fori_loop(0, half, f_body, (jnp.int32(0), jnp.int32(0)))

    sets = (tab, tab2, tab3, tab4)
    for tb in sets:
        def ti_body(j, x, tb=tb):
            tb[pl.ds(j * _L, _L)] = neg_row
            return x
        lax.fori_loop(0, _BPW, ti_body, 0)

    bks[pl.ds(cnt0, _L)] = jnp.full((_L,), lo, jnp.int32)
    keys[pl.ds(cnt0, _L)] = neg_row
    bks[pl.ds(base1 + cnt1, _L)] = jnp.full((_L,), lo, jnp.int32)
    keys[pl.ds(base1 + cnt1, _L)] = neg_row

    def make_ins(base):
        def ins_body(g, x):
            bkv = bks[pl.ds(base + g * _L, _L)]
            kvv = keys[pl.ds(base + g * _L, _L)]
            for j in range(_L):
                tb = sets[j % 4]
                off = (bkv[j] - lo) * _L
                row = tb[pl.ds(off, _L)]
                row = jnp.where(lanes == 0, jnp.maximum(row, kvv[j]), row)
                tb[pl.ds(off, _L)] = lax.sort(row)
            return x
        return ins_body
    lax.fori_loop(0, (cnt0 + _L - 1) // _L, make_ins(0), 0)
    lax.fori_loop(0, (cnt1 + _L - 1) // _L, make_ins(base1), 0)

    def scan_body(p, acc):
        jj = jnp.where(is_a, p, _BPW - 1 - p)
        r01 = _merge16(tab[pl.ds(jj * _L, _L)], tab2[pl.ds(jj * _L, _L)])
        r23 = _merge16(tab3[pl.ds(jj * _L, _L)], tab4[pl.ds(jj * _L, _L)])
        merged = _merge16(acc, _merge16(r01, r23))
        scn[pl.ds(jj * _L, _L)] = jnp.where(is_a, merged, acc)
        return merged
    total = lax.fori_loop(0, _BPW, scan_body, neg_row)

    tot[...] = total
    pltpu.sync_copy(tot, sums_hbm.at[pl.ds((c * _NW + s) * _L, _L)])
    plsc.subcore_barrier()
    pltpu.sync_copy(sums_hbm.at[pl.ds(c * _NW * _L, _NW * _L)], sumv)

    def ca_body(j, acc):
        return _merge16(acc, sumv[pl.ds(j * _L, _L)])
    carry_a = lax.fori_loop(0, s, ca_body, neg_row)
    carry_b = lax.fori_loop(s + 1, _NW, ca_body, neg_row)
    carry = jnp.where(is_a, carry_a, carry_b)

    def ap_body(jj, x):
        blk[jj] = _merge16(scn[pl.ds(jj * _L, _L)], carry)
        return x
    lax.fori_loop(0, _BPW, ap_body, 0)

    @pl.when(is_a)
    def _():
        pltpu.sync_copy(blk, outa_hbm.at[pl.ds(lo, _BPW), :])

    @pl.when(jnp.logical_not(is_a))
    def _():
        pltpu.sync_copy(blk, outb_hbm.at[pl.ds(lo, _BPW), :])


_sc_call = functools.partial(
    pl.kernel,
    out_type=[
        jax.ShapeDtypeStruct((_R, _L), jnp.float32),
        jax.ShapeDtypeStruct((_R, _L), jnp.float32),
        jax.ShapeDtypeStruct((2 * _NW * _L,), jnp.float32),
    ],
    mesh=plsc.VectorSubcoreMesh(core_axis_name="c", subcore_axis_name="s"),
    compiler_params=pltpu.CompilerParams(needs_layout_passes=False),
    scratch_types=[
        pltpu.VMEM((_NPAD,), jnp.float32),
        pltpu.VMEM((_NPAD,), jnp.float32),
        pltpu.VMEM((_NPAD + 2 * _L,), jnp.float32),
        pltpu.VMEM((_NPAD + 2 * _L,), jnp.int32),
        pltpu.VMEM((_BPW * _L,), jnp.float32),
        pltpu.VMEM((_BPW * _L,), jnp.float32),
        pltpu.VMEM((_BPW * _L,), jnp.float32),
        pltpu.VMEM((_BPW * _L,), jnp.float32),
        pltpu.VMEM((_BPW * _L,), jnp.float32),
        pltpu.VMEM((_BPW, _L), jnp.float32),
        pltpu.VMEM((_NW * _L,), jnp.float32),
        pltpu.VMEM((_L,), jnp.float32),
    ],
)


def _insert(accs, v):
    out = []
    for a in accs:
        hi = jnp.maximum(a, v)
        v = jnp.minimum(a, v)
        out.append(hi)
    return out


def _combine_body(pairs_ref, pa_ref, nb_ref, out_ref):
    minb = jnp.min(pairs_ref[:, 0:1])
    maxd = jnp.max(pairs_ref[:, 1:2])
    step = (maxd - minb) / jnp.float32(_R - 1)
    lane = lax.broadcasted_iota(jnp.int32, (1, _R), 1)
    t = minb + step * lane.astype(jnp.float32)
    accs = [jnp.zeros((1, _R), jnp.float32)] * _K
    for k in range(_L - _K, _L):
        accs = _insert(accs, pa_ref[k:k + 1, :] - t)
        accs = _insert(accs, t + nb_ref[k:k + 1, :])
    accs.append(jnp.zeros((8 - _K, _R), jnp.float32))
    out_ref[:, :] = jnp.concatenate(accs, axis=0)


def kernel(pairs):
    n = pairs.shape[0]
    padn = _NPAD - n
    b = jnp.concatenate([pairs[:, 0], jnp.full((padn,), 2.0, jnp.float32)])
    d = jnp.concatenate([pairs[:, 1], jnp.full((padn,), -1.0, jnp.float32)])
    pa, nb, _ = _sc_call(_sc_body)(b, d)
    pa = pa.T
    nb = nb.T
    out = pl.pallas_call(
        _combine_body,
        out_shape=jax.ShapeDtypeStruct((8, _R), jnp.float32),
    )(pairs, pa, nb)
    return out[:_K]


# speedup vs baseline: 1.0250x; 1.0250x over previous
import functools
import jax
import jax.numpy as jnp
from jax import lax
from jax.experimental import pallas as pl
from jax.experimental.pallas import tpu as pltpu
from jax.experimental.pallas import tpu_sc as plsc

_K = 5
_R = 1024
_L = 16
_NPAD = 20480
_NW = 16
_BPW = _R // _NW
_NEG = -1e30


def _merge16(u, v):
    return lax.sort(jnp.maximum(u, lax.rev(v, (0,))))


def _sc_body(b_hbm, d_hbm, outa_hbm, outb_hbm, sums_hbm, mm_hbm,
             bv, dv, keys, bks, tab, tab2, tab3, tab4, scn, blk, sumv, tot,
             mmrow, mmall):
    c = lax.axis_index("c")
    s = lax.axis_index("s")
    is_a = c == 0
    lo = s * _BPW
    lanes = lax.iota(jnp.int32, _L)
    neg_row = jnp.full((_L,), _NEG, jnp.float32)

    pltpu.sync_copy(b_hbm, bv)
    pltpu.sync_copy(d_hbm, dv)

    nmm = _NPAD // _L // _NW
    def mm_body(i, carry):
        mb, md = carry
        return (jnp.minimum(mb, bv[pl.ds((s * nmm + i) * _L, _L)]),
                jnp.maximum(md, dv[pl.ds((s * nmm + i) * _L, _L)]))
    mb, md = lax.fori_loop(0, nmm, mm_body,
                           (jnp.full((_L,), 1e30, jnp.float32), neg_row))
    mmrow[pl.ds(0, _L)] = mb
    mmrow[pl.ds(_L, _L)] = md
    pltpu.sync_copy(mmrow, mm_hbm.at[pl.ds((c * _NW + s) * 2 * _L, 2 * _L)])
    plsc.subcore_barrier()
    pltpu.sync_copy(mm_hbm.at[pl.ds(c * _NW * 2 * _L, _NW * 2 * _L)], mmall)

    def mr_body(k, carry):
        mb, md = carry
        return (jnp.minimum(mb, mmall[pl.ds(k * 2 * _L, _L)]),
                jnp.maximum(md, mmall[pl.ds(k * 2 * _L + _L, _L)]))
    mb, md = lax.fori_loop(0, _NW, mr_body,
                           (jnp.full((_L,), 1e30, jnp.float32), neg_row))
    minb = lax.sort(mb)[0]
    maxd = lax.sort(md)[_L - 1]
    spanv = jnp.full((_L,), maxd - minb, jnp.float32)
    inv_dtv = jnp.where(spanv > 0, jnp.float32(_R - 1) / spanv,
                        jnp.zeros((_L,), jnp.float32))

    half = _NPAD // (2 * _L)
    base1 = _NPAD // 2 + _L

    def f_chunk(i, off, base):
        bb = bv[pl.ds(i * _L, _L)]
        dd = dv[pl.ds(i * _L, _L)]
        u = ((bb + dd) * 0.5 - minb) * inv_dtv
        ti = u.astype(jnp.int32)
        bk = jnp.where(ti.astype(jnp.float32) < u, ti + 1, ti)
        bk = jnp.clip(bk, 0, _R - 1)
        key = jnp.where(is_a, dd, -bb)
        msk = (bk >= lo) & (bk < lo + _BPW)
        mi = msk.astype(jnp.int32)
        cum = plsc.cumsum(mi)
        pos = base + off + cum - mi
        plsc.store_scatter(keys, [pos], key, mask=msk)
        plsc.store_scatter(bks, [pos], bk, mask=msk)
        return off + cum[_L - 1]

    def f_body(i, offs):
        o0 = f_chunk(2 * i, offs[0], 0)
        o1 = f_chunk(2 * i + half, offs[1], base1)
        o0 = f_chunk(2 * i + 1, o0, 0)
        o1 = f_chunk(2 * i + 1 + half, o1, base1)
        return (o0, o1)
    cnt0, cnt1 = lax.fori_loop(0, half // 2, f_body,
                               (jnp.int32(0), jnp.int32(0)))

    sets = (tab, tab2, tab3, tab4)
    for tb in sets:
        def ti_body(j, x, tb=tb):
            tb[pl.ds(j * _L, _L)] = neg_row
            return x
        lax.fori_loop(0, _BPW, ti_body, 0)

    bks[pl.ds(cnt0, _L)] = jnp.full((_L,), lo, jnp.int32)
    keys[pl.ds(cnt0, _L)] = neg_row
    bks[pl.ds(base1 + cnt1, _L)] = jnp.full((_L,), lo, jnp.int32)
    keys[pl.ds(base1 + cnt1, _L)] = neg_row

    def make_ins(base):
        def ins_body(g, x):
            bkv = bks[pl.ds(base + g * _L, _L)]
            kvv = keys[pl.ds(base + g * _L, _L)]
            for j in range(_L):
                tb = sets[j % 4]
                off = (bkv[j] - lo) * _L
                row = tb[pl.ds(off, _L)]
                row = jnp.where(lanes == 0, jnp.maximum(row, kvv[j]), row)
                tb[pl.ds(off, _L)] = lax.sort(row)
            return x
        return ins_body
    lax.fori_loop(0, (cnt0 + _L - 1) // _L, make_ins(0), 0)
    lax.fori_loop(0, (cnt1 + _L - 1) // _L, make_ins(base1), 0)

    def scan_body(p, acc):
        jj = jnp.where(is_a, p, _BPW - 1 - p)
        r01 = _merge16(tab[pl.ds(jj * _L, _L)], tab2[pl.ds(jj * _L, _L)])
        r23 = _merge16(tab3[pl.ds(jj * _L, _L)], tab4[pl.ds(jj * _L, _L)])
        merged = _merge16(acc, _merge16(r01, r23))
        scn[pl.ds(jj * _L, _L)] = jnp.where(is_a, merged, acc)
        return merged
    total = lax.fori_loop(0, _BPW, scan_body, neg_row)

    tot[...] = total
    pltpu.sync_copy(tot, sums_hbm.at[pl.ds((c * _NW + s) * _L, _L)])
    plsc.subcore_barrier()
    pltpu.sync_copy(sums_hbm.at[pl.ds(c * _NW * _L, _NW * _L)], sumv)

    def ca_body(j, acc):
        return _merge16(acc, sumv[pl.ds(j * _L, _L)])
    carry_a = lax.fori_loop(0, s, ca_body, neg_row)
    carry_b = lax.fori_loop(s + 1, _NW, ca_body, neg_row)
    carry = jnp.where(is_a, carry_a, carry_b)

    def ap_body(jj, x):
        blk[jj] = _merge16(scn[pl.ds(jj * _L, _L)], carry)
        return x
    lax.fori_loop(0, _BPW, ap_body, 0)

    @pl.when(is_a)
    def _():
        pltpu.sync_copy(blk, outa_hbm.at[pl.ds(lo, _BPW), :])

    @pl.when(jnp.logical_not(is_a))
    def _():
        pltpu.sync_copy(blk, outb_hbm.at[pl.ds(lo, _BPW), :])


_sc_call = functools.partial(
    pl.kernel,
    out_type=[
        jax.ShapeDtypeStruct((_R, _L), jnp.float32),
        jax.ShapeDtypeStruct((_R, _L), jnp.float32),
        jax.ShapeDtypeStruct((2 * _NW * _L,), jnp.float32),
        jax.ShapeDtypeStruct((2 * _NW * 2 * _L,), jnp.float32),
    ],
    mesh=plsc.VectorSubcoreMesh(core_axis_name="c", subcore_axis_name="s"),
    compiler_params=pltpu.CompilerParams(needs_layout_passes=False),
    scratch_types=[
        pltpu.VMEM((_NPAD,), jnp.float32),
        pltpu.VMEM((_NPAD,), jnp.float32),
        pltpu.VMEM((_NPAD + 2 * _L,), jnp.float32),
        pltpu.VMEM((_NPAD + 2 * _L,), jnp.int32),
        pltpu.VMEM((_BPW * _L,), jnp.float32),
        pltpu.VMEM((_BPW * _L,), jnp.float32),
        pltpu.VMEM((_BPW * _L,), jnp.float32),
        pltpu.VMEM((_BPW * _L,), jnp.float32),
        pltpu.VMEM((_BPW * _L,), jnp.float32),
        pltpu.VMEM((_BPW, _L), jnp.float32),
        pltpu.VMEM((_NW * _L,), jnp.float32),
        pltpu.VMEM((_L,), jnp.float32),
        pltpu.VMEM((2 * _L,), jnp.float32),
        pltpu.VMEM((_NW * 2 * _L,), jnp.float32),
    ],
)


def _insert(accs, v):
    out = []
    for a in accs:
        hi = jnp.maximum(a, v)
        v = jnp.minimum(a, v)
        out.append(hi)
    return out


def _combine_body(pairs_ref, pa_ref, nb_ref, out_ref):
    minb = jnp.min(pairs_ref[:, 0:1])
    maxd = jnp.max(pairs_ref[:, 1:2])
    step = (maxd - minb) / jnp.float32(_R - 1)
    lane = lax.broadcasted_iota(jnp.int32, (1, _R), 1)
    t = minb + step * lane.astype(jnp.float32)
    accs = [jnp.zeros((1, _R), jnp.float32)] * _K
    for k in range(_L - _K, _L):
        accs = _insert(accs, pa_ref[k:k + 1, :] - t)
        accs = _insert(accs, t + nb_ref[k:k + 1, :])
    accs.append(jnp.zeros((8 - _K, _R), jnp.float32))
    out_ref[:, :] = jnp.concatenate(accs, axis=0)


def kernel(pairs):
    n = pairs.shape[0]
    padn = _NPAD - n
    b = jnp.concatenate([pairs[:, 0], jnp.full((padn,), 2.0, jnp.float32)])
    d = jnp.concatenate([pairs[:, 1], jnp.full((padn,), -1.0, jnp.float32)])
    pa, nb, _, _ = _sc_call(_sc_body)(b, d)
    pa = pa.T
    nb = nb.T
    out = pl.pallas_call(
        _combine_body,
        out_shape=jax.ShapeDtypeStruct((8, _R), jnp.float32),
    )(pairs, pa, nb)
    return out[:_K]


# speedup vs baseline: 1.2305x; 1.2006x over previous
import functools
import jax
import jax.numpy as jnp
from jax import lax
from jax.experimental import pallas as pl
from jax.experimental.pallas import tpu as pltpu
from jax.experimental.pallas import tpu_sc as plsc

_K = 5
_R = 1024
_L = 16
_NPAD = 20480
_NW = 16
_BPW = _R // _NW
_NEG = -1e30


def _shift_left(row, idxp1):
    return lax.gather(
        row, idxp1[:, None],
        lax.GatherDimensionNumbers(offset_dims=(), collapsed_slice_dims=(0,),
                                   start_index_map=(0,)),
        slice_sizes=(1,), mode=lax.GatherScatterMode.PROMISE_IN_BOUNDS)


def _merge16(u, v):
    return lax.sort(jnp.maximum(u, lax.rev(v, (0,))))


def _sc_body(b_hbm, d_hbm, outa_hbm, outb_hbm, sums_hbm, mm_hbm,
             bv, dv, keys, bks, tab, tab2, tab3, tab4, scn, blk, sumv, tot,
             mmrow, mmall):
    c = lax.axis_index("c")
    s = lax.axis_index("s")
    is_a = c == 0
    lo = s * _BPW
    lanes = lax.iota(jnp.int32, _L)
    neg_row = jnp.full((_L,), _NEG, jnp.float32)

    pltpu.sync_copy(b_hbm, bv)
    pltpu.sync_copy(d_hbm, dv)

    nmm = _NPAD // _L // _NW
    def mm_body(i, carry):
        mb, md = carry
        return (jnp.minimum(mb, bv[pl.ds((s * nmm + i) * _L, _L)]),
                jnp.maximum(md, dv[pl.ds((s * nmm + i) * _L, _L)]))
    mb, md = lax.fori_loop(0, nmm, mm_body,
                           (jnp.full((_L,), 1e30, jnp.float32), neg_row))
    mmrow[pl.ds(0, _L)] = mb
    mmrow[pl.ds(_L, _L)] = md
    pltpu.sync_copy(mmrow, mm_hbm.at[pl.ds((c * _NW + s) * 2 * _L, 2 * _L)])
    plsc.subcore_barrier()
    pltpu.sync_copy(mm_hbm.at[pl.ds(c * _NW * 2 * _L, _NW * 2 * _L)], mmall)

    def mr_body(k, carry):
        mb, md = carry
        return (jnp.minimum(mb, mmall[pl.ds(k * 2 * _L, _L)]),
                jnp.maximum(md, mmall[pl.ds(k * 2 * _L + _L, _L)]))
    mb, md = lax.fori_loop(0, _NW, mr_body,
                           (jnp.full((_L,), 1e30, jnp.float32), neg_row))
    minb = lax.sort(mb)[0]
    maxd = lax.sort(md)[_L - 1]
    spanv = jnp.full((_L,), maxd - minb, jnp.float32)
    inv_dtv = jnp.where(spanv > 0, jnp.float32(_R - 1) / spanv,
                        jnp.zeros((_L,), jnp.float32))

    half = _NPAD // (2 * _L)
    base1 = _NPAD // 2 + _L

    def f_chunk(i, off, base):
        bb = bv[pl.ds(i * _L, _L)]
        dd = dv[pl.ds(i * _L, _L)]
        u = ((bb + dd) * 0.5 - minb) * inv_dtv
        ti = u.astype(jnp.int32)
        bk = jnp.where(ti.astype(jnp.float32) < u, ti + 1, ti)
        bk = jnp.clip(bk, 0, _R - 1)
        key = jnp.where(is_a, dd, -bb)
        msk = (bk >= lo) & (bk < lo + _BPW)
        mi = msk.astype(jnp.int32)
        cum = plsc.cumsum(mi)
        pos = base + off + cum - mi
        plsc.store_scatter(keys, [pos], key, mask=msk)
        plsc.store_scatter(bks, [pos], bk, mask=msk)
        return off + cum[_L - 1]

    def f_body(i, offs):
        o0 = f_chunk(2 * i, offs[0], 0)
        o1 = f_chunk(2 * i + half, offs[1], base1)
        o0 = f_chunk(2 * i + 1, o0, 0)
        o1 = f_chunk(2 * i + 1 + half, o1, base1)
        return (o0, o1)
    cnt0, cnt1 = lax.fori_loop(0, half // 2, f_body,
                               (jnp.int32(0), jnp.int32(0)))

    sets = (tab, tab2, tab3, tab4)
    for tb in sets:
        def ti_body(j, x, tb=tb):
            tb[pl.ds(j * _L, _L)] = neg_row
            return x
        lax.fori_loop(0, _BPW, ti_body, 0)

    bks[pl.ds(cnt0, _L)] = jnp.full((_L,), lo, jnp.int32)
    keys[pl.ds(cnt0, _L)] = neg_row
    bks[pl.ds(base1 + cnt1, _L)] = jnp.full((_L,), lo, jnp.int32)
    keys[pl.ds(base1 + cnt1, _L)] = neg_row

    idxp1 = jnp.minimum(lanes + 1, _L - 1)
    last = lanes == _L - 1

    def make_ins(base):
        def ins_body(g, x):
            bkv = bks[pl.ds(base + g * _L, _L)]
            kvv = keys[pl.ds(base + g * _L, _L)]
            for j in range(_L):
                tb = sets[j % 4]
                off = (bkv[j] - lo) * _L
                row = tb[pl.ds(off, _L)]
                v = kvv[j]
                shifted = jnp.where(last, jnp.float32(3e38),
                                    _shift_left(row, idxp1))
                row = jnp.where(shifted < v, shifted,
                                jnp.where(row < v, v, row))
                tb[pl.ds(off, _L)] = row
            return x
        return ins_body
    lax.fori_loop(0, (cnt0 + _L - 1) // _L, make_ins(0), 0)
    lax.fori_loop(0, (cnt1 + _L - 1) // _L, make_ins(base1), 0)

    def scan_body(p, acc):
        jj = jnp.where(is_a, p, _BPW - 1 - p)
        r01 = _merge16(tab[pl.ds(jj * _L, _L)], tab2[pl.ds(jj * _L, _L)])
        r23 = _merge16(tab3[pl.ds(jj * _L, _L)], tab4[pl.ds(jj * _L, _L)])
        merged = _merge16(acc, _merge16(r01, r23))
        scn[pl.ds(jj * _L, _L)] = jnp.where(is_a, merged, acc)
        return merged
    total = lax.fori_loop(0, _BPW, scan_body, neg_row)

    tot[...] = total
    pltpu.sync_copy(tot, sums_hbm.at[pl.ds((c * _NW + s) * _L, _L)])
    plsc.subcore_barrier()
    pltpu.sync_copy(sums_hbm.at[pl.ds(c * _NW * _L, _NW * _L)], sumv)

    def ca_body(j, acc):
        return _merge16(acc, sumv[pl.ds(j * _L, _L)])
    carry_a = lax.fori_loop(0, s, ca_body, neg_row)
    carry_b = lax.fori_loop(s + 1, _NW, ca_body, neg_row)
    carry = jnp.where(is_a, carry_a, carry_b)

    def ap_body(jj, x):
        blk[jj] = _merge16(scn[pl.ds(jj * _L, _L)], carry)
        return x
    lax.fori_loop(0, _BPW, ap_body, 0)

    @pl.when(is_a)
    def _():
        pltpu.sync_copy(blk, outa_hbm.at[pl.ds(lo, _BPW), :])

    @pl.when(jnp.logical_not(is_a))
    def _():
        pltpu.sync_copy(blk, outb_hbm.at[pl.ds(lo, _BPW), :])


_sc_call = functools.partial(
    pl.kernel,
    out_type=[
        jax.ShapeDtypeStruct((_R, _L), jnp.float32),
        jax.ShapeDtypeStruct((_R, _L), jnp.float32),
        jax.ShapeDtypeStruct((2 * _NW * _L,), jnp.float32),
        jax.ShapeDtypeStruct((2 * _NW * 2 * _L,), jnp.float32),
    ],
    mesh=plsc.VectorSubcoreMesh(core_axis_name="c", subcore_axis_name="s"),
    compiler_params=pltpu.CompilerParams(needs_layout_passes=False),
    scratch_types=[
        pltpu.VMEM((_NPAD,), jnp.float32),
        pltpu.VMEM((_NPAD,), jnp.float32),
        pltpu.VMEM((_NPAD + 2 * _L,), jnp.float32),
        pltpu.VMEM((_NPAD + 2 * _L,), jnp.int32),
        pltpu.VMEM((_BPW * _L,), jnp.float32),
        pltpu.VMEM((_BPW * _L,), jnp.float32),
        pltpu.VMEM((_BPW * _L,), jnp.float32),
        pltpu.VMEM((_BPW * _L,), jnp.float32),
        pltpu.VMEM((_BPW * _L,), jnp.float32),
        pltpu.VMEM((_BPW, _L), jnp.float32),
        pltpu.VMEM((_NW * _L,), jnp.float32),
        pltpu.VMEM((_L,), jnp.float32),
        pltpu.VMEM((2 * _L,), jnp.float32),
        pltpu.VMEM((_NW * 2 * _L,), jnp.float32),
    ],
)


def _insert(accs, v):
    out = []
    for a in accs:
        hi = jnp.maximum(a, v)
        v = jnp.minimum(a, v)
        out.append(hi)
    return out


def _combine_body(pairs_ref, pa_ref, nb_ref, out_ref):
    minb = jnp.min(pairs_ref[:, 0:1])
    maxd = jnp.max(pairs_ref[:, 1:2])
    step = (maxd - minb) / jnp.float32(_R - 1)
    lane = lax.broadcasted_iota(jnp.int32, (1, _R), 1)
    t = minb + step * lane.astype(jnp.float32)
    accs = [jnp.zeros((1, _R), jnp.float32)] * _K
    for k in range(_L - _K, _L):
        accs = _insert(accs, pa_ref[k:k + 1, :] - t)
        accs = _insert(accs, t + nb_ref[k:k + 1, :])
    accs.append(jnp.zeros((8 - _K, _R), jnp.float32))
    out_ref[:, :] = jnp.concatenate(accs, axis=0)


def kernel(pairs):
    n = pairs.shape[0]
    padn = _NPAD - n
    b = jnp.concatenate([pairs[:, 0], jnp.full((padn,), 2.0, jnp.float32)])
    d = jnp.concatenate([pairs[:, 1], jnp.full((padn,), -1.0, jnp.float32)])
    pa, nb, _, _ = _sc_call(_sc_body)(b, d)
    pa = pa.T
    nb = nb.T
    out = pl.pallas_call(
        _combine_body,
        out_shape=jax.ShapeDtypeStruct((8, _R), jnp.float32),
    )(pairs, pa, nb)
    return out[:_K]
